# baseline (speedup 1.0000x reference)
import functools

import jax
import jax.numpy as jnp
from jax import lax
from jax.experimental import pallas as pl
from jax.experimental.pallas import tpu as pltpu
from jax.experimental.pallas import tpu_sc as plsc

NUM_USERS = 100000
NUM_ITEMS = 1000000
D = 128
B = 16384

NC = 2
NS = 16
NW = NC * NS
BPW = B // NW
CH = 128
NCHUNK = BPW // CH
GRP = 8

_PERM_DN = lax.GatherDimensionNumbers(
    offset_dims=(), collapsed_slice_dims=(0,), start_index_map=(0,))


def _vperm(x, idx):
    return lax.gather(x, idx[:, None], _PERM_DN, slice_sizes=(1,),
                      mode=lax.GatherScatterMode.PROMISE_IN_BOUNDS)


def _gmf_body(uid_hbm, iid_hbm, ut_hbm, it_hbm, w_hbm, out_hbm,
              uid_v, iid_v, urows, irows, w_v, logit_v, out_v, sems):
    wid = lax.axis_index("s") * NC + lax.axis_index("c")
    base = wid * BPW
    pltpu.sync_copy(uid_hbm.at[pl.ds(base, BPW)], uid_v)
    pltpu.sync_copy(iid_hbm.at[pl.ds(base, BPW)], iid_v)
    pltpu.sync_copy(w_hbm, w_v)
    w_regs = [w_v[pl.ds(j * 16, 16)] for j in range(D // 16)]
    zero = jnp.zeros((16,), jnp.float32)
    lane = lax.iota(jnp.int32, 16)
    lo_half = lane < 8
    lane7 = lane & 7
    unscramble = (lane & 1) * 8 + ((lane & 7) >> 1) + ((lane >> 3) << 2)
    for g in range(BPW // 16):
        logit_v[pl.ds(g * 16, 16)] = zero

    def start(c, buf):
        cu = pltpu.async_copy(ut_hbm.at[uid_v.at[pl.ds(c * CH, CH)]],
                              urows.at[buf], sems.at[buf, 0])
        ci = pltpu.async_copy(it_hbm.at[iid_v.at[pl.ds(c * CH, CH)]],
                              irows.at[buf], sems.at[buf, 1])
        return cu, ci

    def wait(c, buf):
        pltpu.make_async_copy(ut_hbm.at[uid_v.at[pl.ds(c * CH, CH)]],
                              urows.at[buf], sems.at[buf, 0]).wait()
        pltpu.make_async_copy(it_hbm.at[iid_v.at[pl.ds(c * CH, CH)]],
                              irows.at[buf], sems.at[buf, 1]).wait()

    def compute(c, buf):
        u_c = urows.at[buf]
        i_c = irows.at[buf]

        def block(t, carry):
            q = (t % 2) * 4
            s = zero
            for p in range(4):
                halves = []
                for e2 in (0, 1):
                    b = t * GRP + 2 * p + e2
                    acc = zero
                    for j in range(D // 16):
                        ks = pl.ds(j * 16, 16)
                        acc = acc + u_c[b, ks] * i_c[b, ks] * w_regs[j]
                    halves.append(acc + _vperm(acc, lane ^ 8))
                z = jnp.where(lo_half, halves[0], halves[1])
                for sh in (4, 2, 1):
                    z = z + _vperm(z, lane ^ sh)
                s = jnp.where(lane7 == p + q, z, s)
            dst = pl.ds(c * CH + (t // 2) * 16, 16)
            logit_v[dst] = logit_v[dst] + s
            return carry

        lax.fori_loop(0, CH // GRP, block, 0)

    start(0, 0)

    def pair(cc, carry):
        c0 = 2 * cc
        start(c0 + 1, 1)
        wait(c0, 0)
        compute(c0, 0)

        @pl.when(c0 + 2 < NCHUNK)
        def _():
            start(c0 + 2, 0)

        wait(c0 + 1, 1)
        compute(c0 + 1, 1)
        return carry

    lax.fori_loop(0, NCHUNK // 2, pair, 0)

    for g in range(BPW // 16):
        s = _vperm(logit_v[pl.ds(g * 16, 16)], unscramble)
        out_v[pl.ds(g * 16, 16)] = 1.0 / (1.0 + jnp.exp(-s))

    pltpu.sync_copy(out_v, out_hbm.at[pl.ds(base, BPW)])


@functools.partial(
    pl.kernel,
    out_type=jax.ShapeDtypeStruct((B,), jnp.float32),
    mesh=plsc.VectorSubcoreMesh(core_axis_name="c", subcore_axis_name="s"),
    scratch_types=[
        pltpu.VMEM((BPW,), jnp.int32),
        pltpu.VMEM((BPW,), jnp.int32),
        pltpu.VMEM((2, CH, D), jnp.float32),
        pltpu.VMEM((2, CH, D), jnp.float32),
        pltpu.VMEM((D,), jnp.float32),
        pltpu.VMEM((BPW,), jnp.float32),
        pltpu.VMEM((BPW,), jnp.float32),
        pltpu.SemaphoreType.DMA((2, 2)),
    ],
)
def _gmf(uid_hbm, iid_hbm, ut_hbm, it_hbm, w_hbm, out_hbm,
         uid_v, iid_v, urows, irows, w_v, logit_v, out_v, sems):
    _gmf_body(uid_hbm, iid_hbm, ut_hbm, it_hbm, w_hbm, out_hbm,
              uid_v, iid_v, urows, irows, w_v, logit_v, out_v, sems)


def kernel(user_ids, item_ids, embed_user, embed_item, fc_w):
    return _gmf(user_ids.astype(jnp.int32), item_ids.astype(jnp.int32),
                embed_user, embed_item, fc_w.reshape(D))


# speedup vs baseline: 1.0313x; 1.0313x over previous
import functools

import jax
import jax.numpy as jnp
from jax import lax
from jax.experimental import pallas as pl
from jax.experimental.pallas import tpu as pltpu
from jax.experimental.pallas import tpu_sc as plsc

NUM_USERS = 100000
NUM_ITEMS = 1000000
D = 128
B = 16384

NC = 2
NS = 16
NW = NC * NS
BPW = B // NW
CH = 128
NCHUNK = BPW // CH
GRP = 8

_PERM_DN = lax.GatherDimensionNumbers(
    offset_dims=(), collapsed_slice_dims=(0,), start_index_map=(0,))


def _vperm(x, idx):
    return lax.gather(x, idx[:, None], _PERM_DN, slice_sizes=(1,),
                      mode=lax.GatherScatterMode.PROMISE_IN_BOUNDS)


def _gmf_body(uid_hbm, iid_hbm, ut_hbm, it_hbm, w_hbm, out_hbm,
              uid_v, iid_v, urows, irows, w_v, logit_v, out_v, sems):
    wid = lax.axis_index("s") * NC + lax.axis_index("c")
    base = wid * BPW
    pltpu.sync_copy(uid_hbm.at[pl.ds(base, BPW)], uid_v)
    pltpu.sync_copy(iid_hbm.at[pl.ds(base, BPW)], iid_v)
    pltpu.sync_copy(w_hbm, w_v)
    w_regs = [w_v[pl.ds(j * 16, 16)] for j in range(D // 16)]
    zero = jnp.zeros((16,), jnp.float32)
    lane = lax.iota(jnp.int32, 16)
    lo_half = lane < 8
    lane7 = lane & 7
    unscramble = (lane & 1) * 8 + ((lane & 7) >> 1) + ((lane >> 3) << 2)
    for g in range(BPW // 16):
        logit_v[pl.ds(g * 16, 16)] = zero

    def start(c, buf):
        cu = pltpu.async_copy(ut_hbm.at[uid_v.at[pl.ds(c * CH, CH)]],
                              urows.at[buf], sems.at[buf, 0])
        ci = pltpu.async_copy(it_hbm.at[iid_v.at[pl.ds(c * CH, CH)]],
                              irows.at[buf], sems.at[buf, 1])
        return cu, ci

    def wait(c, buf):
        pltpu.make_async_copy(ut_hbm.at[uid_v.at[pl.ds(c * CH, CH)]],
                              urows.at[buf], sems.at[buf, 0]).wait()
        pltpu.make_async_copy(it_hbm.at[iid_v.at[pl.ds(c * CH, CH)]],
                              irows.at[buf], sems.at[buf, 1]).wait()

    def compute(c, buf):
        u_c = urows.at[buf]
        i_c = irows.at[buf]

        def block(t, carry):
            off = (t % 2) * GRP
            s = zero
            for e in range(GRP):
                b = t * GRP + e
                acc = zero
                for j in range(D // 16):
                    ks = pl.ds(j * 16, 16)
                    acc = acc + u_c[b, ks] * i_c[b, ks] * w_regs[j]
                for sh in (8, 4, 2, 1):
                    acc = acc + _vperm(acc, lane ^ sh)
                s = jnp.where(lane == off + e, acc, s)
            dst = pl.ds(c * CH + (t // 2) * 16, 16)
            logit_v[dst] = logit_v[dst] + s
            return carry

        lax.fori_loop(0, CH // GRP, block, 0)

    start(0, 0)

    def one(c, carry):
        buf = c % 2

        @pl.when(c + 1 < NCHUNK)
        def _():
            start(c + 1, 1 - buf)

        wait(c, buf)
        compute(c, buf)
        return carry

    lax.fori_loop(0, NCHUNK, one, 0)

    for g in range(BPW // 16):
        s = logit_v[pl.ds(g * 16, 16)]
        out_v[pl.ds(g * 16, 16)] = 1.0 / (1.0 + jnp.exp(-s))

    pltpu.sync_copy(out_v, out_hbm.at[pl.ds(base, BPW)])


@functools.partial(
    pl.kernel,
    out_type=jax.ShapeDtypeStruct((B,), jnp.float32),
    mesh=plsc.VectorSubcoreMesh(core_axis_name="c", subcore_axis_name="s"),
    scratch_types=[
        pltpu.VMEM((BPW,), jnp.int32),
        pltpu.VMEM((BPW,), jnp.int32),
        pltpu.VMEM((2, CH, D), jnp.float32),
        pltpu.VMEM((2, CH, D), jnp.float32),
        pltpu.VMEM((D,), jnp.float32),
        pltpu.VMEM((BPW,), jnp.float32),
        pltpu.VMEM((BPW,), jnp.float32),
        pltpu.SemaphoreType.DMA((2, 2)),
    ],
)
def _gmf(uid_hbm, iid_hbm, ut_hbm, it_hbm, w_hbm, out_hbm,
         uid_v, iid_v, urows, irows, w_v, logit_v, out_v, sems):
    _gmf_body(uid_hbm, iid_hbm, ut_hbm, it_hbm, w_hbm, out_hbm,
              uid_v, iid_v, urows, irows, w_v, logit_v, out_v, sems)


def kernel(user_ids, item_ids, embed_user, embed_item, fc_w):
    return _gmf(user_ids.astype(jnp.int32), item_ids.astype(jnp.int32),
                embed_user, embed_item, fc_w.reshape(D))


# speedup vs baseline: 1.0777x; 1.0450x over previous
import functools

import jax
import jax.numpy as jnp
from jax import lax
from jax.experimental import pallas as pl
from jax.experimental.pallas import tpu as pltpu
from jax.experimental.pallas import tpu_sc as plsc

NUM_USERS = 100000
NUM_ITEMS = 1000000
D = 128
B = 16384

NC = 2
NS = 16
NW = NC * NS
BPW = B // NW
CH = 128
NCHUNK = BPW // CH
GRP = 8

_PERM_DN = lax.GatherDimensionNumbers(
    offset_dims=(), collapsed_slice_dims=(0,), start_index_map=(0,))


def _vperm(x, idx):
    return lax.gather(x, idx[:, None], _PERM_DN, slice_sizes=(1,),
                      mode=lax.GatherScatterMode.PROMISE_IN_BOUNDS)


def _gmf_body(uid_hbm, iid_hbm, ut_hbm, it_hbm, w_hbm, out_hbm,
              uid_v, iid_v, urows, irows, w_v, logit_v, out_v, sems, sems2):
    wid = lax.axis_index("s") * NC + lax.axis_index("c")
    base = wid * BPW
    cu0 = pltpu.async_copy(uid_hbm.at[pl.ds(base, BPW)], uid_v, sems2.at[0])
    ci0 = pltpu.async_copy(iid_hbm.at[pl.ds(base, BPW)], iid_v, sems2.at[1])
    cw0 = pltpu.async_copy(w_hbm, w_v, sems2.at[2])
    zero = jnp.zeros((16,), jnp.float32)
    lane = lax.iota(jnp.int32, 16)

    def zero_blk(g, carry):
        logit_v[pl.ds(g * 16, 16)] = zero
        return carry

    lax.fori_loop(0, BPW // 16, zero_blk, 0)
    cu0.wait()
    ci0.wait()
    cw0.wait()
    w_regs = [w_v[pl.ds(j * 16, 16)] for j in range(D // 16)]

    def start(c, buf):
        cu = pltpu.async_copy(ut_hbm.at[uid_v.at[pl.ds(c * CH, CH)]],
                              urows.at[buf], sems.at[buf, 0])
        ci = pltpu.async_copy(it_hbm.at[iid_v.at[pl.ds(c * CH, CH)]],
                              irows.at[buf], sems.at[buf, 1])
        return cu, ci

    def wait(c, buf):
        pltpu.make_async_copy(ut_hbm.at[uid_v.at[pl.ds(c * CH, CH)]],
                              urows.at[buf], sems.at[buf, 0]).wait()
        pltpu.make_async_copy(it_hbm.at[iid_v.at[pl.ds(c * CH, CH)]],
                              irows.at[buf], sems.at[buf, 1]).wait()

    def compute(c, buf):
        u_c = urows.at[buf]
        i_c = irows.at[buf]

        def block(t, carry):
            off = (t % 2) * GRP
            s = zero
            for e in range(GRP):
                b = t * GRP + e
                acc = zero
                for j in range(D // 16):
                    ks = pl.ds(j * 16, 16)
                    acc = acc + u_c[b, ks] * i_c[b, ks] * w_regs[j]
                for sh in (8, 4, 2, 1):
                    acc = acc + _vperm(acc, lane ^ sh)
                s = jnp.where(lane == off + e, acc, s)
            dst = pl.ds(c * CH + (t // 2) * 16, 16)
            logit_v[dst] = logit_v[dst] + s
            return carry

        lax.fori_loop(0, CH // GRP, block, 0)

    start(0, 0)

    def one(c, carry):
        buf = c % 2

        @pl.when(c + 1 < NCHUNK)
        def _():
            start(c + 1, 1 - buf)

        wait(c, buf)
        compute(c, buf)
        return carry

    lax.fori_loop(0, NCHUNK, one, 0)

    def sig_blk(g, carry):
        s = logit_v[pl.ds(g * 16, 16)]
        out_v[pl.ds(g * 16, 16)] = 1.0 / (1.0 + jnp.exp(-s))
        return carry

    lax.fori_loop(0, BPW // 16, sig_blk, 0)

    pltpu.sync_copy(out_v, out_hbm.at[pl.ds(base, BPW)])


@functools.partial(
    pl.kernel,
    out_type=jax.ShapeDtypeStruct((B,), jnp.float32),
    mesh=plsc.VectorSubcoreMesh(core_axis_name="c", subcore_axis_name="s"),
    scratch_types=[
        pltpu.VMEM((BPW,), jnp.int32),
        pltpu.VMEM((BPW,), jnp.int32),
        pltpu.VMEM((2, CH, D), jnp.float32),
        pltpu.VMEM((2, CH, D), jnp.float32),
        pltpu.VMEM((D,), jnp.float32),
        pltpu.VMEM((BPW,), jnp.float32),
        pltpu.VMEM((BPW,), jnp.float32),
        pltpu.SemaphoreType.DMA((2, 2)),
        pltpu.SemaphoreType.DMA((3,)),
    ],
)
def _gmf(uid_hbm, iid_hbm, ut_hbm, it_hbm, w_hbm, out_hbm,
         uid_v, iid_v, urows, irows, w_v, logit_v, out_v, sems, sems2):
    _gmf_body(uid_hbm, iid_hbm, ut_hbm, it_hbm, w_hbm, out_hbm,
              uid_v, iid_v, urows, irows, w_v, logit_v, out_v, sems, sems2)


def kernel(user_ids, item_ids, embed_user, embed_item, fc_w):
    return _gmf(user_ids.astype(jnp.int32), item_ids.astype(jnp.int32),
                embed_user, embed_item, fc_w.reshape(D))


# speedup vs baseline: 1.1627x; 1.0788x over previous
import functools

import jax
import jax.numpy as jnp
from jax import lax
from jax.experimental import pallas as pl
from jax.experimental.pallas import tpu as pltpu
from jax.experimental.pallas import tpu_sc as plsc

NUM_USERS = 100000
NUM_ITEMS = 1000000
D = 128
B = 16384

NC = 2
NS = 16
NW = NC * NS
BPW = B // NW
CH = 128
NCHUNK = BPW // CH
GRP = 8

_PERM_DN = lax.GatherDimensionNumbers(
    offset_dims=(), collapsed_slice_dims=(0,), start_index_map=(0,))


def _vperm(x, idx):
    return lax.gather(x, idx[:, None], _PERM_DN, slice_sizes=(1,),
                      mode=lax.GatherScatterMode.PROMISE_IN_BOUNDS)


def _gmf_body(uid_hbm, iid_hbm, ut_hbm, it_hbm, w_hbm, out_hbm,
              uid_v, iid_v, urows, irows, w_v, logit_v, out_v, sems, sems2):
    wid = lax.axis_index("s") * NC + lax.axis_index("c")
    base = wid * BPW
    cu0 = pltpu.async_copy(uid_hbm.at[pl.ds(base, BPW)], uid_v, sems2.at[0])
    ci0 = pltpu.async_copy(iid_hbm.at[pl.ds(base, BPW)], iid_v, sems2.at[1])
    cw0 = pltpu.async_copy(w_hbm, w_v, sems2.at[2])
    zero = jnp.zeros((16,), jnp.float32)
    lane = lax.iota(jnp.int32, 16)

    def zero_blk(g, carry):
        logit_v[pl.ds(g * 16, 16)] = zero
        return carry

    lax.fori_loop(0, BPW // 16, zero_blk, 0)
    cu0.wait()
    ci0.wait()
    cw0.wait()
    w_regs = [w_v[pl.ds(j * 16, 16)] for j in range(D // 16)]

    def start(c, buf):
        cu = pltpu.async_copy(ut_hbm.at[uid_v.at[pl.ds(c * CH, CH)]],
                              urows.at[buf], sems.at[buf, 0])
        ci = pltpu.async_copy(it_hbm.at[iid_v.at[pl.ds(c * CH, CH)]],
                              irows.at[buf], sems.at[buf, 1])
        return cu, ci

    def wait(c, buf):
        pltpu.make_async_copy(ut_hbm.at[uid_v.at[pl.ds(c * CH, CH)]],
                              urows.at[buf], sems.at[buf, 0]).wait()
        pltpu.make_async_copy(it_hbm.at[iid_v.at[pl.ds(c * CH, CH)]],
                              irows.at[buf], sems.at[buf, 1]).wait()

    start(0, 0)

    def step(t, carry):
        c = t // (CH // GRP)
        tt = t % (CH // GRP)
        buf = c % 2

        @pl.when(tt == 0)
        def _():
            @pl.when(c + 1 < NCHUNK)
            def __():
                start(c + 1, 1 - buf)

            wait(c, buf)

        u_c = urows.at[buf]
        i_c = irows.at[buf]
        off = (tt % 2) * GRP
        s = zero
        for e in range(GRP):
            b = tt * GRP + e
            acc = zero
            for j in range(D // 16):
                ks = pl.ds(j * 16, 16)
                acc = acc + u_c[b, ks] * i_c[b, ks] * w_regs[j]
            for sh in (8, 4, 2, 1):
                acc = acc + _vperm(acc, lane ^ sh)
            s = jnp.where(lane == off + e, acc, s)
        dst = pl.ds(c * CH + (tt // 2) * 16, 16)
        logit_v[dst] = logit_v[dst] + s
        return carry

    lax.fori_loop(0, NCHUNK * (CH // GRP), step, 0)

    def sig_blk(g, carry):
        s = logit_v[pl.ds(g * 16, 16)]
        out_v[pl.ds(g * 16, 16)] = 1.0 / (1.0 + jnp.exp(-s))
        return carry

    lax.fori_loop(0, BPW // 16, sig_blk, 0)

    pltpu.sync_copy(out_v, out_hbm.at[pl.ds(base, BPW)])


@functools.partial(
    pl.kernel,
    out_type=jax.ShapeDtypeStruct((B,), jnp.float32),
    mesh=plsc.VectorSubcoreMesh(core_axis_name="c", subcore_axis_name="s"),
    scratch_types=[
        pltpu.VMEM((BPW,), jnp.int32),
        pltpu.VMEM((BPW,), jnp.int32),
        pltpu.VMEM((2, CH, D), jnp.float32),
        pltpu.VMEM((2, CH, D), jnp.float32),
        pltpu.VMEM((D,), jnp.float32),
        pltpu.VMEM((BPW,), jnp.float32),
        pltpu.VMEM((BPW,), jnp.float32),
        pltpu.SemaphoreType.DMA((2, 2)),
        pltpu.SemaphoreType.DMA((3,)),
    ],
)
def _gmf(uid_hbm, iid_hbm, ut_hbm, it_hbm, w_hbm, out_hbm,
         uid_v, iid_v, urows, irows, w_v, logit_v, out_v, sems, sems2):
    _gmf_body(uid_hbm, iid_hbm, ut_hbm, it_hbm, w_hbm, out_hbm,
              uid_v, iid_v, urows, irows, w_v, logit_v, out_v, sems, sems2)


def kernel(user_ids, item_ids, embed_user, embed_item, fc_w):
    return _gmf(user_ids.astype(jnp.int32), item_ids.astype(jnp.int32),
                embed_user, embed_item, fc_w.reshape(D))
